# baseline (device time: 32274 ns/iter reference)
import jax
import jax.numpy as jnp
from jax import lax
from jax.experimental import pallas as pl
from jax.experimental.pallas import tpu as pltpu

N_DEV = 16
PLANE = 4
ZDIM = 4
NA = 512
BP = 2


def kernel(x, w_mat):
    m, k_per = x.shape
    _, n = w_mat.shape
    m_blk = m // N_DEV
    na2 = NA // 2
    nb = n - NA
    nb2 = nb // 2
    sup = m // ZDIM
    piece = sup // BP

    def body(x_ref, w_ref, out_ref, p_ref,
             xr1s, xr1r, xr2s, xr2r, yr1s, yr1r, yr2s, yr2r,
             b1us, b1ds, b1ur, b1dr, a2s, a2r, b2s, b2r,
             sem_xr1s, sem_xr1r, sem_xr2s, sem_xr2r,
             sem_yr1s, sem_yr1r, sem_yr2s, sem_yr2r,
             sem_b1us, sem_b1ds, sem_b1ur, sem_b1dr,
             sem_a2s, sem_a2r, sem_b2s, sem_b2r):
        my = lax.axis_index("i")
        z = my // PLANE
        p = my % PLANE

        def pz(v):
            return jnp.where(v == 2, 3, jnp.where(v == 3, 2, v))

        zeta = pz(z)
        xpart = p + 1 - 2 * (p % 2)
        ypart = 3 - p
        diagp = 3 - xpart
        dev_x = z * PLANE + xpart
        dev_y = z * PLANE + ypart
        dev_diag = z * PLANE + diagp
        z_up = pz((zeta + 1) % ZDIM) * PLANE + p
        z_dn = pz((zeta - 1) % ZDIM) * PLANE + p
        z_far = pz((zeta + 2) % ZDIM) * PLANE + p

        cs_a = pl.ds(0, na2)
        cs_b = pl.ds(na2, na2)
        cols_b_u = pl.ds(NA, nb2)
        cols_b_d = pl.ds(NA + nb2, nb2)

        def rc(send, recv, ssem, rsem, idx, dev):
            return pltpu.make_async_remote_copy(
                src_ref=send.at[idx], dst_ref=recv.at[idx],
                send_sem=ssem.at[idx], recv_sem=rsem.at[idx],
                device_id=(dev,), device_id_type=pl.DeviceIdType.MESH,
            )

        def rc2(send, recv, ssem, rsem, i0, i1, dev):
            return pltpu.make_async_remote_copy(
                src_ref=send.at[i0, i1], dst_ref=recv.at[i0, i1],
                send_sem=ssem.at[i0, i1], recv_sem=rsem.at[i0, i1],
                device_id=(dev,), device_id_type=pl.DeviceIdType.MESH,
            )

        def rc_plain(send, recv, ssem, rsem, dev):
            return pltpu.make_async_remote_copy(
                src_ref=send, dst_ref=recv, send_sem=ssem, recv_sem=rsem,
                device_id=(dev,), device_id_type=pl.DeviceIdType.MESH,
            )

        def rc_from_p(rows, cols, recv, ssem, rsem, i0, i1, dev):
            return pltpu.make_async_remote_copy(
                src_ref=p_ref.at[rows, cols], dst_ref=recv.at[i0, i1],
                send_sem=ssem.at[i0, i1], recv_sem=rsem.at[i0, i1],
                device_id=(dev,), device_id_type=pl.DeviceIdType.MESH,
            )

        p_ref[...] = jnp.dot(
            x_ref[...], w_ref[...], preferred_element_type=jnp.float32
        )

        barrier_sem = pltpu.get_barrier_semaphore()
        for nbr in (dev_x, dev_y, dev_diag, z_up, z_dn, z_far):
            pl.semaphore_signal(
                barrier_sem, inc=1,
                device_id=(nbr,), device_id_type=pl.DeviceIdType.MESH,
            )
        pl.semaphore_wait(barrier_sem, 6)

        for slot, qx, qy in ((0, diagp, diagp), (1, xpart, ypart)):
            for k in range(ZDIM):
                j = (z + 1 + k) % ZDIM
                rows_x = pl.ds((j * PLANE + qx) * m_blk, m_blk)
                rows_y = pl.ds((j * PLANE + qy) * m_blk, m_blk)
                rc_from_p(rows_x, cs_a, xr1r, sem_xr1s, sem_xr1r,
                          slot, j, dev_x).start()
                rc_from_p(rows_y, cs_b, yr1r, sem_yr1s, sem_yr1r,
                          slot, j, dev_y).start()

        ju_b = pz((zeta - 1) % ZDIM)
        jd_b = pz((zeta + 1) % ZDIM)
        for h in range(BP):
            rc_from_p(pl.ds(ju_b * sup + h * piece, piece), cols_b_u,
                      b1ur, sem_b1us, sem_b1ur, 0, h, z_up).start()
            rc_from_p(pl.ds(jd_b * sup + h * piece, piece), cols_b_d,
                      b1dr, sem_b1ds, sem_b1dr, 0, h, z_dn).start()

        for k in range(ZDIM):
            j = (z + 1 + k) % ZDIM
            rc2(xr1s, xr1r, sem_xr1s, sem_xr1r, 0, j, dev_x).wait_recv()
            rows = pl.ds((j * PLANE + ypart) * m_blk, m_blk)
            xr2s[j, :, :] = p_ref[rows, cs_a] + xr1r[0, j, :, :]
            rc(xr2s, xr2r, sem_xr2s, sem_xr2r, j, dev_y).start()
            rc2(yr1s, yr1r, sem_yr1s, sem_yr1r, 0, j, dev_y).wait_recv()
            rows = pl.ds((j * PLANE + xpart) * m_blk, m_blk)
            yr2s[j, :, :] = p_ref[rows, cs_b] + yr1r[0, j, :, :]
            rc(yr2s, yr2r, sem_yr2s, sem_yr2r, j, dev_x).start()

        for s in range(1, 3):
            ju_b = pz((zeta - s - 1) % ZDIM)
            jd_b = pz((zeta + s + 1) % ZDIM)
            for h in range(BP):
                rows_bu = pl.ds(ju_b * sup + h * piece, piece)
                rows_bd = pl.ds(jd_b * sup + h * piece, piece)
                rc2(b1us, b1ur, sem_b1us, sem_b1ur, s - 1, h, z_up).wait_recv()
                rc2(b1ds, b1dr, sem_b1ds, sem_b1dr, s - 1, h, z_dn).wait_recv()
                b1us[s, h, :, :] = p_ref[rows_bu, cols_b_u] + b1ur[s - 1, h, :, :]
                b1ds[s, h, :, :] = p_ref[rows_bd, cols_b_d] + b1dr[s - 1, h, :, :]
                rc2(b1us, b1ur, sem_b1us, sem_b1ur, s, h, z_up).start()
                rc2(b1ds, b1dr, sem_b1ds, sem_b1dr, s, h, z_dn).start()

        for delta in range(1, ZDIM):
            j = (z + delta) % ZDIM
            rc2(xr1s, xr1r, sem_xr1s, sem_xr1r, 1, j, dev_x).wait_recv()
            rc2(yr1s, yr1r, sem_yr1s, sem_yr1r, 1, j, dev_y).wait_recv()
            rc(xr2s, xr2r, sem_xr2s, sem_xr2r, j, dev_y).wait_recv()
            rc(yr2s, yr2r, sem_yr2s, sem_yr2r, j, dev_x).wait_recv()
            tgt = j * PLANE + p
            rows_j = pl.ds((j * PLANE + p) * m_blk, m_blk)
            a2s[delta - 1, :, 0:na2] = (
                p_ref[rows_j, cs_a] + xr1r[1, j, :, :] + xr2r[j, :, :])
            a2s[delta - 1, :, na2:NA] = (
                p_ref[rows_j, cs_b] + yr1r[1, j, :, :] + yr2r[j, :, :])
            rc(a2s, a2r, sem_a2s, sem_a2r, delta - 1, tgt).start()

        for h in range(BP):
            rc2(b1us, b1ur, sem_b1us, sem_b1ur, 2, h, z_up).wait_recv()
            rc2(b1ds, b1dr, sem_b1ds, sem_b1dr, 2, h, z_dn).wait_recv()
        for delta in range(1, PLANE):
            q = (p + delta) % PLANE
            tgt = z * PLANE + q
            rows_q = pl.ds((z * PLANE + q) * m_blk, m_blk)
            hq = q // BP
            oq = (q % BP) * m_blk
            b2s[delta - 1, :, 0:nb2] = (
                p_ref[rows_q, cols_b_u] + b1ur[2, hq, pl.ds(oq, m_blk), :])
            b2s[delta - 1, :, nb2:nb] = (
                p_ref[rows_q, cols_b_d] + b1dr[2, hq, pl.ds(oq, m_blk), :])
            rc(b2s, b2r, sem_b2s, sem_b2r, delta - 1, tgt).start()

        rc2(xr1s, xr1r, sem_xr1s, sem_xr1r, 1, z, dev_x).wait_recv()
        rc2(yr1s, yr1r, sem_yr1s, sem_yr1r, 1, z, dev_y).wait_recv()
        rc(xr2s, xr2r, sem_xr2s, sem_xr2r, z, dev_y).wait_recv()
        rc(yr2s, yr2r, sem_yr2s, sem_yr2r, z, dev_x).wait_recv()
        for d in range(3):
            rc(a2s, a2r, sem_a2s, sem_a2r, d, my).wait_recv()
            rc(b2s, b2r, sem_b2s, sem_b2r, d, my).wait_recv()
        rows_m = pl.ds(my * m_blk, m_blk)
        hp = p // BP
        op = (p % BP) * m_blk
        yar = (p_ref[rows_m, cs_a] + xr1r[1, z, :, :] + xr2r[z, :, :]
               + a2r[0, :, 0:na2] + a2r[1, :, 0:na2] + a2r[2, :, 0:na2])
        yal = (p_ref[rows_m, cs_b] + yr1r[1, z, :, :] + yr2r[z, :, :]
               + a2r[0, :, na2:NA] + a2r[1, :, na2:NA] + a2r[2, :, na2:NA])
        ybu = (p_ref[rows_m, cols_b_u] + b1ur[2, hp, pl.ds(op, m_blk), :]
               + b2r[0, :, 0:nb2] + b2r[1, :, 0:nb2] + b2r[2, :, 0:nb2])
        ybd = (p_ref[rows_m, cols_b_d] + b1dr[2, hp, pl.ds(op, m_blk), :]
               + b2r[0, :, nb2:nb] + b2r[1, :, nb2:nb] + b2r[2, :, nb2:nb])
        out_ref[:, cs_a] = yar * (1.0 / (1.0 + jnp.exp(-yar)))
        out_ref[:, cs_b] = yal * (1.0 / (1.0 + jnp.exp(-yal)))
        out_ref[:, cols_b_u] = ybu * (1.0 / (1.0 + jnp.exp(-ybu)))
        out_ref[:, cols_b_d] = ybd * (1.0 / (1.0 + jnp.exp(-ybd)))

        for slot in range(2):
            for j in range(ZDIM):
                rc2(xr1s, xr1r, sem_xr1s, sem_xr1r, slot, j, dev_x).wait_send()
                rc2(yr1s, yr1r, sem_yr1s, sem_yr1r, slot, j, dev_y).wait_send()
        for j in range(ZDIM):
            rc(xr2s, xr2r, sem_xr2s, sem_xr2r, j, dev_y).wait_send()
            rc(yr2s, yr2r, sem_yr2s, sem_yr2r, j, dev_x).wait_send()
        for s in range(3):
            for h in range(BP):
                rc2(b1us, b1ur, sem_b1us, sem_b1ur, s, h, z_up).wait_send()
                rc2(b1ds, b1dr, sem_b1ds, sem_b1dr, s, h, z_dn).wait_send()
            rc(a2s, a2r, sem_a2s, sem_a2r, s, my).wait_send()
            rc(b2s, b2r, sem_b2s, sem_b2r, s, my).wait_send()

    out_shape = jax.ShapeDtypeStruct((m_blk, n), jnp.float32)
    dma = pltpu.SemaphoreType.DMA
    return pl.pallas_call(
        body,
        out_shape=out_shape,
        in_specs=[
            pl.BlockSpec(memory_space=pltpu.VMEM),
            pl.BlockSpec(memory_space=pltpu.VMEM),
        ],
        out_specs=pl.BlockSpec(memory_space=pltpu.VMEM),
        scratch_shapes=[
            pltpu.VMEM((m, n), jnp.float32),
            pltpu.VMEM((2, ZDIM, m_blk, na2), jnp.float32),
            pltpu.VMEM((2, ZDIM, m_blk, na2), jnp.float32),
            pltpu.VMEM((ZDIM, m_blk, na2), jnp.float32),
            pltpu.VMEM((ZDIM, m_blk, na2), jnp.float32),
            pltpu.VMEM((2, ZDIM, m_blk, na2), jnp.float32),
            pltpu.VMEM((2, ZDIM, m_blk, na2), jnp.float32),
            pltpu.VMEM((ZDIM, m_blk, na2), jnp.float32),
            pltpu.VMEM((ZDIM, m_blk, na2), jnp.float32),
            pltpu.VMEM((3, BP, piece, nb2), jnp.float32),
            pltpu.VMEM((3, BP, piece, nb2), jnp.float32),
            pltpu.VMEM((3, BP, piece, nb2), jnp.float32),
            pltpu.VMEM((3, BP, piece, nb2), jnp.float32),
            pltpu.VMEM((3, m_blk, NA), jnp.float32),
            pltpu.VMEM((3, m_blk, NA), jnp.float32),
            pltpu.VMEM((3, m_blk, nb), jnp.float32),
            pltpu.VMEM((3, m_blk, nb), jnp.float32),
            dma((2, ZDIM)), dma((2, ZDIM)), dma((ZDIM,)), dma((ZDIM,)),
            dma((2, ZDIM)), dma((2, ZDIM)), dma((ZDIM,)), dma((ZDIM,)),
            dma((3, BP)), dma((3, BP)), dma((3, BP)), dma((3, BP)),
            dma((3,)), dma((3,)), dma((3,)), dma((3,)),
        ],
        compiler_params=pltpu.CompilerParams(collective_id=0),
    )(x, w_mat)


# device time: 31951 ns/iter; 1.0101x vs baseline; 1.0101x over previous
import jax
import jax.numpy as jnp
from jax import lax
from jax.experimental import pallas as pl
from jax.experimental.pallas import tpu as pltpu

N_DEV = 16
PLANE = 4
ZDIM = 4
NA = 512
BP = 2


def kernel(x, w_mat):
    m, k_per = x.shape
    _, n = w_mat.shape
    m_blk = m // N_DEV
    na2 = NA // 2
    nb = n - NA
    nb2 = nb // 2
    sup = m // ZDIM
    piece = sup // BP

    def body(x_ref, w_ref, out_ref, p_ref,
             xr1s, xr1r, xr2s, xr2r, yr1s, yr1r, yr2s, yr2r,
             b1us, b1ds, b1ur, b1dr, a2s, a2r, b2s, b2r,
             sem_xr1s, sem_xr1r, sem_xr2s, sem_xr2r,
             sem_yr1s, sem_yr1r, sem_yr2s, sem_yr2r,
             sem_b1us, sem_b1ds, sem_b1ur, sem_b1dr,
             sem_a2s, sem_a2r, sem_b2s, sem_b2r):
        my = lax.axis_index("i")
        z = my // PLANE
        p = my % PLANE

        def pz(v):
            return jnp.where(v == 2, 3, jnp.where(v == 3, 2, v))

        zeta = pz(z)
        xpart = p + 1 - 2 * (p % 2)
        ypart = 3 - p
        diagp = 3 - xpart
        dev_x = z * PLANE + xpart
        dev_y = z * PLANE + ypart
        dev_diag = z * PLANE + diagp
        z_up = pz((zeta + 1) % ZDIM) * PLANE + p
        z_dn = pz((zeta - 1) % ZDIM) * PLANE + p
        z_far = pz((zeta + 2) % ZDIM) * PLANE + p

        cs_a = pl.ds(0, na2)
        cs_b = pl.ds(na2, na2)
        cols_b_u = pl.ds(NA, nb2)
        cols_b_d = pl.ds(NA + nb2, nb2)

        def rc(send, recv, ssem, rsem, idx, dev):
            return pltpu.make_async_remote_copy(
                src_ref=send.at[idx], dst_ref=recv.at[idx],
                send_sem=ssem.at[idx], recv_sem=rsem.at[idx],
                device_id=(dev,), device_id_type=pl.DeviceIdType.MESH,
            )

        def rc2(send, recv, ssem, rsem, i0, i1, dev):
            return pltpu.make_async_remote_copy(
                src_ref=send.at[i0, i1], dst_ref=recv.at[i0, i1],
                send_sem=ssem.at[i0, i1], recv_sem=rsem.at[i0, i1],
                device_id=(dev,), device_id_type=pl.DeviceIdType.MESH,
            )

        def rc_plain(send, recv, ssem, rsem, dev):
            return pltpu.make_async_remote_copy(
                src_ref=send, dst_ref=recv, send_sem=ssem, recv_sem=rsem,
                device_id=(dev,), device_id_type=pl.DeviceIdType.MESH,
            )

        def rc_from_p(rows, cols, recv, ssem, rsem, i0, i1, dev):
            return pltpu.make_async_remote_copy(
                src_ref=p_ref.at[rows, cols], dst_ref=recv.at[i0, i1],
                send_sem=ssem.at[i0, i1], recv_sem=rsem.at[i0, i1],
                device_id=(dev,), device_id_type=pl.DeviceIdType.MESH,
            )

        p_ref[...] = jnp.dot(
            x_ref[...], w_ref[...], preferred_element_type=jnp.float32
        )

        barrier_sem = pltpu.get_barrier_semaphore()
        for nbr in (dev_x, dev_y, dev_diag, z_up, z_dn, z_far):
            pl.semaphore_signal(
                barrier_sem, inc=1,
                device_id=(nbr,), device_id_type=pl.DeviceIdType.MESH,
            )
        pl.semaphore_wait(barrier_sem, 6)

        for slot, q in ((0, diagp), (1, xpart)):
            for j in range(ZDIM):
                rows = pl.ds((j * PLANE + q) * m_blk, m_blk)
                xr1s[slot, j, :, :] = p_ref[rows, cs_a]
            rc(xr1s, xr1r, sem_xr1s, sem_xr1r, slot, dev_x).start()
        for slot, q in ((0, diagp), (1, ypart)):
            for j in range(ZDIM):
                rows = pl.ds((j * PLANE + q) * m_blk, m_blk)
                yr1s[slot, j, :, :] = p_ref[rows, cs_b]
            rc(yr1s, yr1r, sem_yr1s, sem_yr1r, slot, dev_y).start()

        ju_b = pz((zeta - 1) % ZDIM)
        jd_b = pz((zeta + 1) % ZDIM)
        for h in range(BP):
            rc_from_p(pl.ds(ju_b * sup + h * piece, piece), cols_b_u,
                      b1ur, sem_b1us, sem_b1ur, 0, h, z_up).start()
            rc_from_p(pl.ds(jd_b * sup + h * piece, piece), cols_b_d,
                      b1dr, sem_b1ds, sem_b1dr, 0, h, z_dn).start()

        rc(xr1s, xr1r, sem_xr1s, sem_xr1r, 0, dev_x).wait_recv()
        for j in range(ZDIM):
            rows = pl.ds((j * PLANE + ypart) * m_blk, m_blk)
            xr2s[j, :, :] = p_ref[rows, cs_a] + xr1r[0, j, :, :]
        rc_plain(xr2s, xr2r, sem_xr2s, sem_xr2r, dev_y).start()
        rc(yr1s, yr1r, sem_yr1s, sem_yr1r, 0, dev_y).wait_recv()
        for j in range(ZDIM):
            rows = pl.ds((j * PLANE + xpart) * m_blk, m_blk)
            yr2s[j, :, :] = p_ref[rows, cs_b] + yr1r[0, j, :, :]
        rc_plain(yr2s, yr2r, sem_yr2s, sem_yr2r, dev_x).start()

        for s in range(1, 3):
            ju_b = pz((zeta - s - 1) % ZDIM)
            jd_b = pz((zeta + s + 1) % ZDIM)
            for h in range(BP):
                rows_bu = pl.ds(ju_b * sup + h * piece, piece)
                rows_bd = pl.ds(jd_b * sup + h * piece, piece)
                rc2(b1us, b1ur, sem_b1us, sem_b1ur, s - 1, h, z_up).wait_recv()
                rc2(b1ds, b1dr, sem_b1ds, sem_b1dr, s - 1, h, z_dn).wait_recv()
                b1us[s, h, :, :] = p_ref[rows_bu, cols_b_u] + b1ur[s - 1, h, :, :]
                b1ds[s, h, :, :] = p_ref[rows_bd, cols_b_d] + b1dr[s - 1, h, :, :]
                rc2(b1us, b1ur, sem_b1us, sem_b1ur, s, h, z_up).start()
                rc2(b1ds, b1dr, sem_b1ds, sem_b1dr, s, h, z_dn).start()

        rc(xr1s, xr1r, sem_xr1s, sem_xr1r, 1, dev_x).wait_recv()
        rc(yr1s, yr1r, sem_yr1s, sem_yr1r, 1, dev_y).wait_recv()
        rc_plain(xr2s, xr2r, sem_xr2s, sem_xr2r, dev_y).wait_recv()
        rc_plain(yr2s, yr2r, sem_yr2s, sem_yr2r, dev_x).wait_recv()
        for delta in range(1, ZDIM):
            j = (z + delta) % ZDIM
            tgt = j * PLANE + p
            rows_j = pl.ds((j * PLANE + p) * m_blk, m_blk)
            a2s[delta - 1, :, 0:na2] = (
                p_ref[rows_j, cs_a] + xr1r[1, j, :, :] + xr2r[j, :, :])
            a2s[delta - 1, :, na2:NA] = (
                p_ref[rows_j, cs_b] + yr1r[1, j, :, :] + yr2r[j, :, :])
            rc(a2s, a2r, sem_a2s, sem_a2r, delta - 1, tgt).start()

        for h in range(BP):
            rc2(b1us, b1ur, sem_b1us, sem_b1ur, 2, h, z_up).wait_recv()
            rc2(b1ds, b1dr, sem_b1ds, sem_b1dr, 2, h, z_dn).wait_recv()
        for delta in range(1, PLANE):
            q = (p + delta) % PLANE
            tgt = z * PLANE + q
            rows_q = pl.ds((z * PLANE + q) * m_blk, m_blk)
            hq = q // BP
            oq = (q % BP) * m_blk
            b2s[delta - 1, :, 0:nb2] = (
                p_ref[rows_q, cols_b_u] + b1ur[2, hq, pl.ds(oq, m_blk), :])
            b2s[delta - 1, :, nb2:nb] = (
                p_ref[rows_q, cols_b_d] + b1dr[2, hq, pl.ds(oq, m_blk), :])
            rc(b2s, b2r, sem_b2s, sem_b2r, delta - 1, tgt).start()

        for d in range(3):
            rc(a2s, a2r, sem_a2s, sem_a2r, d, my).wait_recv()
            rc(b2s, b2r, sem_b2s, sem_b2r, d, my).wait_recv()
        rows_m = pl.ds(my * m_blk, m_blk)
        hp = p // BP
        op = (p % BP) * m_blk
        yar = (p_ref[rows_m, cs_a] + xr1r[1, z, :, :] + xr2r[z, :, :]
               + a2r[0, :, 0:na2] + a2r[1, :, 0:na2] + a2r[2, :, 0:na2])
        yal = (p_ref[rows_m, cs_b] + yr1r[1, z, :, :] + yr2r[z, :, :]
               + a2r[0, :, na2:NA] + a2r[1, :, na2:NA] + a2r[2, :, na2:NA])
        ybu = (p_ref[rows_m, cols_b_u] + b1ur[2, hp, pl.ds(op, m_blk), :]
               + b2r[0, :, 0:nb2] + b2r[1, :, 0:nb2] + b2r[2, :, 0:nb2])
        ybd = (p_ref[rows_m, cols_b_d] + b1dr[2, hp, pl.ds(op, m_blk), :]
               + b2r[0, :, nb2:nb] + b2r[1, :, nb2:nb] + b2r[2, :, nb2:nb])
        out_ref[:, cs_a] = yar * (1.0 / (1.0 + jnp.exp(-yar)))
        out_ref[:, cs_b] = yal * (1.0 / (1.0 + jnp.exp(-yal)))
        out_ref[:, cols_b_u] = ybu * (1.0 / (1.0 + jnp.exp(-ybu)))
        out_ref[:, cols_b_d] = ybd * (1.0 / (1.0 + jnp.exp(-ybd)))

        for slot in range(2):
            rc(xr1s, xr1r, sem_xr1s, sem_xr1r, slot, dev_x).wait_send()
            rc(yr1s, yr1r, sem_yr1s, sem_yr1r, slot, dev_y).wait_send()
        rc_plain(xr2s, xr2r, sem_xr2s, sem_xr2r, dev_y).wait_send()
        rc_plain(yr2s, yr2r, sem_yr2s, sem_yr2r, dev_x).wait_send()
        for s in range(3):
            for h in range(BP):
                rc2(b1us, b1ur, sem_b1us, sem_b1ur, s, h, z_up).wait_send()
                rc2(b1ds, b1dr, sem_b1ds, sem_b1dr, s, h, z_dn).wait_send()
            rc(a2s, a2r, sem_a2s, sem_a2r, s, my).wait_send()
            rc(b2s, b2r, sem_b2s, sem_b2r, s, my).wait_send()

    out_shape = jax.ShapeDtypeStruct((m_blk, n), jnp.float32)
    dma = pltpu.SemaphoreType.DMA
    return pl.pallas_call(
        body,
        out_shape=out_shape,
        in_specs=[
            pl.BlockSpec(memory_space=pltpu.VMEM),
            pl.BlockSpec(memory_space=pltpu.VMEM),
        ],
        out_specs=pl.BlockSpec(memory_space=pltpu.VMEM),
        scratch_shapes=[
            pltpu.VMEM((m, n), jnp.float32),
            pltpu.VMEM((2, ZDIM, m_blk, na2), jnp.float32),
            pltpu.VMEM((2, ZDIM, m_blk, na2), jnp.float32),
            pltpu.VMEM((ZDIM, m_blk, na2), jnp.float32),
            pltpu.VMEM((ZDIM, m_blk, na2), jnp.float32),
            pltpu.VMEM((2, ZDIM, m_blk, na2), jnp.float32),
            pltpu.VMEM((2, ZDIM, m_blk, na2), jnp.float32),
            pltpu.VMEM((ZDIM, m_blk, na2), jnp.float32),
            pltpu.VMEM((ZDIM, m_blk, na2), jnp.float32),
            pltpu.VMEM((3, BP, piece, nb2), jnp.float32),
            pltpu.VMEM((3, BP, piece, nb2), jnp.float32),
            pltpu.VMEM((3, BP, piece, nb2), jnp.float32),
            pltpu.VMEM((3, BP, piece, nb2), jnp.float32),
            pltpu.VMEM((3, m_blk, NA), jnp.float32),
            pltpu.VMEM((3, m_blk, NA), jnp.float32),
            pltpu.VMEM((3, m_blk, nb), jnp.float32),
            pltpu.VMEM((3, m_blk, nb), jnp.float32),
            dma((2,)), dma((2,)), dma, dma,
            dma((2,)), dma((2,)), dma, dma,
            dma((3, BP)), dma((3, BP)), dma((3, BP)), dma((3, BP)),
            dma((3,)), dma((3,)), dma((3,)), dma((3,)),
        ],
        compiler_params=pltpu.CompilerParams(collective_id=0),
    )(x, w_mat)
